# baseline (speedup 1.0000x reference)
import functools

import jax
import jax.numpy as jnp
from jax import lax
from jax.experimental import pallas as pl
from jax.experimental.pallas import tpu as pltpu
from jax.experimental.pallas import tpu_sc as plsc

N = 10000
D = 128
E = 320000
EL = 200000

NC = 2
NS = 16
NW = NC * NS

NR = 10240
STRIPE = NR // NS
EPW = 10240
EPAD = NW * EPW

ELW = 6528

SEG_B = 64
SEG_SLOTS = 5
ELPAD = NW * ELW

_MESH = dict(core_axis_name="c", subcore_axis_name="s",
             num_cores=NC, num_subcores=NS)


def _seg_sum_body(with_gather, *refs):
    if with_gather:
        src_hbm, dst_hbm, x_hbm, z_hbm, agg_out = refs[:5]
        rest = refs[5:]
        slots = [tuple(rest[q * 3:q * 3 + 3]) + (rest[3 * SEG_SLOTS + 1 + q],)
                 for q in range(SEG_SLOTS)]
        agg_sp = rest[3 * SEG_SLOTS]
        rows_w = slots[0][2]
        stage_n = SEG_B
    else:
        (dst_hbm, z_hbm, agg_out,
         didx_w, rows_w, agg_sp, sem) = refs
        stage_n = 128

    c = lax.axis_index("c")
    s = lax.axis_index("s")
    w = s * NC + c

    r0 = s * STRIPE
    for q in range(STRIPE // stage_n):
        pltpu.sync_copy(z_hbm.at[pl.ds(r0 + q * stage_n, stage_n)], rows_w)
        pltpu.sync_copy(rows_w, agg_sp.at[pl.ds(r0 + q * stage_n, stage_n)])
    plsc.subcore_barrier()

    base = w * EPW
    if not with_gather:
        ones = jnp.ones((16,), jnp.float32)

        def fill(i, carry):
            for j in range(D // 16):
                rows_w[i, pl.ds(j * 16, 16)] = ones
            return carry

        lax.fori_loop(0, 128, fill, 0)

    if with_gather:
        nburst = EPW // SEG_B
        nslot = len(slots)

        def fire(slot, off):
            sidx, didx, rows, sem = slot
            pltpu.sync_copy(src_hbm.at[pl.ds(off, SEG_B)], sidx)
            pltpu.sync_copy(dst_hbm.at[pl.ds(off, SEG_B)], didx)
            pltpu.async_copy(x_hbm.at[sidx], rows, sem)

        for q, slot in enumerate(slots):
            fire(slot, base + q * SEG_B)

        def rnd(p, carry):
            for q, slot in enumerate(slots):
                sidx, didx, rows, sem = slot
                i = p * nslot + q
                pltpu.make_async_copy(x_hbm.at[sidx], rows, sem).wait()
                pltpu.sync_copy(rows, agg_sp.at[didx], add=True)
                fire(slot, base + lax.rem(i + nslot, nburst) * SEG_B)
            return carry

        lax.fori_loop(0, nburst // nslot, rnd, 0)
        for slot in slots:
            sidx, _, rows, sem = slot
            pltpu.make_async_copy(x_hbm.at[sidx], rows, sem).wait()
    else:
        def burst(i, carry):
            off = base + i * 128
            pltpu.sync_copy(dst_hbm.at[pl.ds(off, 128)], didx_w)
            pltpu.sync_copy(rows_w, agg_sp.at[didx_w], add=True)
            return carry

        lax.fori_loop(0, EPW // 128, burst, 0)
    plsc.subcore_barrier()

    out_base = c * NR + r0
    for q in range(STRIPE // stage_n):
        pltpu.sync_copy(agg_sp.at[pl.ds(r0 + q * stage_n, stage_n)], rows_w)
        pltpu.sync_copy(rows_w, agg_out.at[pl.ds(out_base + q * stage_n, stage_n)])


def _make_seg_sum(with_gather):
    if with_gather:
        scratch = [
            pltpu.VMEM((SEG_B,), jnp.int32),
            pltpu.VMEM((SEG_B,), jnp.int32),
            pltpu.VMEM((SEG_B, D), jnp.float32),
        ] * SEG_SLOTS + [
            pltpu.VMEM_SHARED((NR, D), jnp.float32),
        ] + [pltpu.SemaphoreType.DMA] * SEG_SLOTS
    else:
        scratch = [
            pltpu.VMEM((128,), jnp.int32),
            pltpu.VMEM((128, D), jnp.float32),
            pltpu.VMEM_SHARED((NR, D), jnp.float32),
            pltpu.SemaphoreType.DMA,
        ]
    return pl.kernel(
        functools.partial(_seg_sum_body, with_gather),
        out_type=jax.ShapeDtypeStruct((NC * NR, D), jnp.float32),
        mesh=plsc.VectorSubcoreMesh(**_MESH),
        scratch_types=scratch,
    )


def _cls_body(h_hbm, e0_hbm, e1_hbm, pred_out,
              i0_a, i1_a, a_a, b_a, i0_b, i1_b, a_b, b_b,
              i0_c, i1_c, a_c, b_c,
              out_v, sem_a, sem_b, sem_c):
    c = lax.axis_index("c")
    s = lax.axis_index("s")
    w = s * NC + c
    base = w * ELW
    nburst = ELW // 128
    lanes = lax.iota(jnp.int32, 16)
    slots = [
        (i0_a, i1_a, a_a, b_a, sem_a),
        (i0_b, i1_b, a_b, b_b, sem_b),
        (i0_c, i1_c, a_c, b_c, sem_c),
    ]
    nslot = len(slots)

    def fire(slot, off):
        i0_w, i1_w, a_v, b_v, sem = slot
        pltpu.sync_copy(e0_hbm.at[pl.ds(off, 128)], i0_w)
        pltpu.sync_copy(e1_hbm.at[pl.ds(off, 128)], i1_w)
        pltpu.async_copy(h_hbm.at[i0_w], a_v, sem)
        pltpu.async_copy(h_hbm.at[i1_w], b_v, sem)

    def drain(slot):
        i0_w, i1_w, a_v, b_v, sem = slot
        pltpu.make_async_copy(h_hbm.at[i0_w], a_v, sem).wait()
        pltpu.make_async_copy(h_hbm.at[i1_w], b_v, sem).wait()

    def compute(slot, off):
        _, _, a_v, b_v, _ = slot

        def grp(g, carry2):
            res = jnp.zeros((16,), jnp.float32)
            for r16 in range(16):
                r = g * 16 + r16
                acc = a_v[r, pl.ds(0, 16)] * b_v[r, pl.ds(0, 16)]
                for j in range(1, D // 16):
                    acc = acc + (a_v[r, pl.ds(j * 16, 16)]
                                 * b_v[r, pl.ds(j * 16, 16)])
                for k in (8, 4, 2, 1):
                    acc = acc + jnp.take(acc, lanes ^ k)
                res = jnp.where(lanes == r16, acc, res)
            out_v[pl.ds(g * 16, 16)] = res
            return carry2

        lax.fori_loop(0, 8, grp, 0)
        pltpu.sync_copy(out_v, pred_out.at[pl.ds(off, 128)])

    for q, slot in enumerate(slots):
        fire(slot, base + q * 128)

    def rnd(p, carry):
        for q, slot in enumerate(slots):
            i = p * nslot + q
            off = base + i * 128
            drain(slot)
            compute(slot, off)
            fire(slot, base + lax.rem(i + nslot, nburst) * 128)
        return carry

    lax.fori_loop(0, nburst // nslot, rnd, 0)
    for slot in slots:
        drain(slot)


_cls_kernel = pl.kernel(
    _cls_body,
    out_type=jax.ShapeDtypeStruct((ELPAD,), jnp.float32),
    mesh=plsc.VectorSubcoreMesh(**_MESH),
    scratch_types=[
        pltpu.VMEM((128,), jnp.int32),
        pltpu.VMEM((128,), jnp.int32),
        pltpu.VMEM((128, D), jnp.float32),
        pltpu.VMEM((128, D), jnp.float32),
    ] * 3 + [
        pltpu.VMEM((128,), jnp.float32),
    ] + [pltpu.SemaphoreType.DMA] * 3,
)


def _tc_body(relu, agg_ref, cnt_ref, x_ref, wl_ref, wr_ref, bl_ref, out_ref):
    aggs = agg_ref[0] + agg_ref[1]
    cnt = cnt_ref[0, :, 0:1] + cnt_ref[1, :, 0:1]
    mean = aggs / jnp.maximum(cnt, 1.0)
    h = lax.dot_general(mean, wl_ref[...], (((1,), (1,)), ((), ())),
                        preferred_element_type=jnp.float32)
    h = h + bl_ref[...]
    h = h + lax.dot_general(x_ref[...], wr_ref[...], (((1,), (1,)), ((), ())),
                            preferred_element_type=jnp.float32)
    if relu:
        h = jnp.maximum(h, 0.0)
    out_ref[...] = h


def _tc_layer(relu, agg, cnt, x, wl, wr, bl):
    R = 1000
    grid = (N // R,)
    return pl.pallas_call(
        functools.partial(_tc_body, relu),
        grid=grid,
        in_specs=[
            pl.BlockSpec((NC, R, D), lambda i: (0, i, 0)),
            pl.BlockSpec((NC, R, D), lambda i: (0, i, 0)),
            pl.BlockSpec((R, D), lambda i: (i, 0)),
            pl.BlockSpec((D, D), lambda i: (0, 0)),
            pl.BlockSpec((D, D), lambda i: (0, 0)),
            pl.BlockSpec((1, D), lambda i: (0, 0)),
        ],
        out_specs=pl.BlockSpec((R, D), lambda i: (i, 0)),
        out_shape=jax.ShapeDtypeStruct((N, D), jnp.float32),
    )(agg, cnt, x, wl, wr, bl)


_seg_sum = _make_seg_sum(True)
_cnt_sum = _make_seg_sum(False)


def kernel(x, edge_index, edge_label_index, Wl1, bl1, Wr1, Wl2, bl2, Wr2):
    ei = edge_index.astype(jnp.int32)
    eli = edge_label_index.astype(jnp.int32)

    pad = EPAD - E
    src = jnp.concatenate([ei[0], jnp.zeros((pad,), jnp.int32)])
    dst = jnp.concatenate([ei[1], jnp.full((pad,), N, jnp.int32)])

    z128 = jnp.zeros((NR, D), jnp.float32)

    agg1 = _seg_sum(src, dst, x, z128).reshape(NC, NR, D)
    cnt = _cnt_sum(dst, z128).reshape(NC, NR, D)
    h1 = _tc_layer(True, agg1, cnt, x, Wl1, Wr1, bl1.reshape(1, D))
    agg2 = _seg_sum(src, dst, h1, z128).reshape(NC, NR, D)
    h2 = _tc_layer(False, agg2, cnt, h1, Wl2, Wr2, bl2.reshape(1, D))

    lpad = ELPAD - EL
    e0 = jnp.concatenate([eli[0], jnp.zeros((lpad,), jnp.int32)])
    e1 = jnp.concatenate([eli[1], jnp.zeros((lpad,), jnp.int32)])
    pred = _cls_kernel(h2, e0, e1)
    return pred[:EL]


# speedup vs baseline: 1.2095x; 1.2095x over previous
import functools

import jax
import jax.numpy as jnp
from jax import lax
from jax.experimental import pallas as pl
from jax.experimental.pallas import tpu as pltpu
from jax.experimental.pallas import tpu_sc as plsc

N = 10000
D = 128
E = 320000
EL = 200000

NC = 2
NS = 16
NW = NC * NS

NR = 10240
STRIPE = NR // NS
EPW = 10240
EPAD = NW * EPW

ELW = 6272
ELPAD = NW * ELW

_MESH = dict(core_axis_name="c", subcore_axis_name="s",
             num_cores=NC, num_subcores=NS)


def _seg_sum_body(with_gather, *refs):
    if with_gather:
        (src_hbm, dst_hbm, x_hbm, z_hbm, agg_out,
         sidx_w, didx_w, rows_w, agg_sp, sem) = refs
    else:
        (dst_hbm, z_hbm, agg_out,
         didx_w, rows_w, agg_sp, sem) = refs
        sidx_w = None

    c = lax.axis_index("c")
    s = lax.axis_index("s")
    w = s * NC + c

    r0 = s * STRIPE
    for q in range(STRIPE // 128):
        pltpu.sync_copy(z_hbm.at[pl.ds(r0 + q * 128, 128)], rows_w)
        pltpu.sync_copy(rows_w, agg_sp.at[pl.ds(r0 + q * 128, 128)])
    plsc.subcore_barrier()

    base = w * EPW
    if not with_gather:
        ones = jnp.ones((16,), jnp.float32)

        def fill(i, carry):
            for j in range(D // 16):
                rows_w[i, pl.ds(j * 16, 16)] = ones
            return carry

        lax.fori_loop(0, 128, fill, 0)

    def batch(i, carry):
        off = w * (EPW // 128) + i * 8
        pltpu.sync_copy(dst_hbm.at[pl.ds(off, 8)], didx_w)
        if with_gather:
            pltpu.sync_copy(src_hbm.at[pl.ds(off, 8)], sidx_w)
        for j in range(8):
            if with_gather:
                pltpu.async_copy(x_hbm.at[sidx_w.at[j, :]], rows_w, sem).wait()
            pltpu.sync_copy(rows_w, agg_sp.at[didx_w.at[j, :]], add=True)
        return carry

    lax.fori_loop(0, EPW // 1024, batch, 0)
    plsc.subcore_barrier()

    out_base = c * NR + r0
    for q in range(STRIPE // 128):
        pltpu.sync_copy(agg_sp.at[pl.ds(r0 + q * 128, 128)], rows_w)
        pltpu.sync_copy(rows_w, agg_out.at[pl.ds(out_base + q * 128, 128)])


def _make_seg_sum(with_gather):
    if with_gather:
        scratch = [
            pltpu.VMEM((8, 128), jnp.int32),
            pltpu.VMEM((8, 128), jnp.int32),
            pltpu.VMEM((128, D), jnp.float32),
            pltpu.VMEM_SHARED((NR, D), jnp.float32),
            pltpu.SemaphoreType.DMA,
        ]
    else:
        scratch = [
            pltpu.VMEM((8, 128), jnp.int32),
            pltpu.VMEM((128, D), jnp.float32),
            pltpu.VMEM_SHARED((NR, D), jnp.float32),
            pltpu.SemaphoreType.DMA,
        ]
    return pl.kernel(
        functools.partial(_seg_sum_body, with_gather),
        out_type=jax.ShapeDtypeStruct((NC * NR, D), jnp.float32),
        mesh=plsc.VectorSubcoreMesh(**_MESH),
        scratch_types=scratch,
    )


def _cls_body(h_hbm, e0_hbm, e1_hbm, pred_out,
              i0_w, i1_w, a_v, b_v, out_v, sem):
    c = lax.axis_index("c")
    s = lax.axis_index("s")
    w = s * NC + c
    base = w * ELW
    lanes = lax.iota(jnp.int32, 16)

    def burst(t, carry):
        off = base + t * 128
        pltpu.sync_copy(e0_hbm.at[pl.ds(off, 128)], i0_w)
        pltpu.sync_copy(e1_hbm.at[pl.ds(off, 128)], i1_w)
        d0 = pltpu.async_copy(h_hbm.at[i0_w], a_v, sem)
        d1 = pltpu.async_copy(h_hbm.at[i1_w], b_v, sem)
        d0.wait()
        d1.wait()

        def grp(g, carry2):
            res = jnp.zeros((16,), jnp.float32)
            for r16 in range(16):
                r = g * 16 + r16
                acc = a_v[r, pl.ds(0, 16)] * b_v[r, pl.ds(0, 16)]
                for j in range(1, D // 16):
                    acc = acc + (a_v[r, pl.ds(j * 16, 16)]
                                 * b_v[r, pl.ds(j * 16, 16)])
                for k in (8, 4, 2, 1):
                    acc = acc + jnp.take(acc, lanes ^ k)
                res = jnp.where(lanes == r16, acc, res)
            out_v[pl.ds(g * 16, 16)] = res
            return carry2

        lax.fori_loop(0, 8, grp, 0)
        pltpu.sync_copy(out_v, pred_out.at[pl.ds(off, 128)])
        return carry

    lax.fori_loop(0, ELW // 128, burst, 0)


_cls_kernel = pl.kernel(
    _cls_body,
    out_type=jax.ShapeDtypeStruct((ELPAD,), jnp.float32),
    mesh=plsc.VectorSubcoreMesh(**_MESH),
    scratch_types=[
        pltpu.VMEM((128,), jnp.int32),
        pltpu.VMEM((128,), jnp.int32),
        pltpu.VMEM((128, D), jnp.float32),
        pltpu.VMEM((128, D), jnp.float32),
        pltpu.VMEM((128,), jnp.float32),
        pltpu.SemaphoreType.DMA,
    ],
)


def _tc_body(relu, agg_ref, cnt_ref, x_ref, wl_ref, wr_ref, bl_ref, out_ref):
    aggs = agg_ref[0] + agg_ref[1]
    cnt = cnt_ref[0, :, 0:1] + cnt_ref[1, :, 0:1]
    mean = aggs / jnp.maximum(cnt, 1.0)
    h = lax.dot_general(mean, wl_ref[...], (((1,), (1,)), ((), ())),
                        preferred_element_type=jnp.float32)
    h = h + bl_ref[...]
    h = h + lax.dot_general(x_ref[...], wr_ref[...], (((1,), (1,)), ((), ())),
                            preferred_element_type=jnp.float32)
    if relu:
        h = jnp.maximum(h, 0.0)
    out_ref[...] = h


def _tc_layer(relu, agg, cnt, x, wl, wr, bl):
    R = 1000
    grid = (N // R,)
    return pl.pallas_call(
        functools.partial(_tc_body, relu),
        grid=grid,
        in_specs=[
            pl.BlockSpec((NC, R, D), lambda i: (0, i, 0)),
            pl.BlockSpec((NC, R, D), lambda i: (0, i, 0)),
            pl.BlockSpec((R, D), lambda i: (i, 0)),
            pl.BlockSpec((D, D), lambda i: (0, 0)),
            pl.BlockSpec((D, D), lambda i: (0, 0)),
            pl.BlockSpec((1, D), lambda i: (0, 0)),
        ],
        out_specs=pl.BlockSpec((R, D), lambda i: (i, 0)),
        out_shape=jax.ShapeDtypeStruct((N, D), jnp.float32),
    )(agg, cnt, x, wl, wr, bl)


_seg_sum = _make_seg_sum(True)
_cnt_sum = _make_seg_sum(False)


def kernel(x, edge_index, edge_label_index, Wl1, bl1, Wr1, Wl2, bl2, Wr2):
    ei = edge_index.astype(jnp.int32)
    eli = edge_label_index.astype(jnp.int32)

    pad = EPAD - E
    src = jnp.concatenate([ei[0], jnp.zeros((pad,), jnp.int32)])
    dst = jnp.concatenate([ei[1], jnp.full((pad,), N, jnp.int32)])

    z128 = jnp.zeros((NR, D), jnp.float32)

    src2 = src.reshape(EPAD // 128, 128)
    dst2 = dst.reshape(EPAD // 128, 128)
    agg1 = _seg_sum(src2, dst2, x, z128).reshape(NC, NR, D)
    cnt = _cnt_sum(dst2, z128).reshape(NC, NR, D)
    h1 = _tc_layer(True, agg1, cnt, x, Wl1, Wr1, bl1.reshape(1, D))
    agg2 = _seg_sum(src2, dst2, h1, z128).reshape(NC, NR, D)
    h2 = _tc_layer(False, agg2, cnt, h1, Wl2, Wr2, bl2.reshape(1, D))

    lpad = ELPAD - EL
    e0 = jnp.concatenate([eli[0], jnp.zeros((lpad,), jnp.int32)])
    e1 = jnp.concatenate([eli[1], jnp.zeros((lpad,), jnp.int32)])
    pred = _cls_kernel(h2, e0, e1)
    return pred[:EL]


# speedup vs baseline: 1.2840x; 1.0616x over previous
import functools

import jax
import jax.numpy as jnp
from jax import lax
from jax.experimental import pallas as pl
from jax.experimental.pallas import tpu as pltpu
from jax.experimental.pallas import tpu_sc as plsc

N = 10000
D = 128
E = 320000
EL = 200000

NC = 2
NS = 16
NW = NC * NS

NR = 10240
STRIPE = NR // NS
EPW = 10240
EPAD = NW * EPW

ELW = 6272
ELPAD = NW * ELW

_MESH = dict(core_axis_name="c", subcore_axis_name="s",
             num_cores=NC, num_subcores=NS)


def _seg_sum_body(with_gather, *refs):
    if with_gather:
        (src_hbm, dst_hbm, x_hbm, z_hbm, agg_out,
         sidx_w, didx_w, rows_w, rows_b, agg_sp, sem, sem_b) = refs
    else:
        (dst_hbm, z_hbm, agg_out,
         didx_w, rows_w, agg_sp, sem) = refs
        sidx_w = None

    c = lax.axis_index("c")
    s = lax.axis_index("s")
    w = s * NC + c

    r0 = s * STRIPE
    for q in range(STRIPE // 128):
        pltpu.sync_copy(z_hbm.at[pl.ds(r0 + q * 128, 128)], rows_w)
        pltpu.sync_copy(rows_w, agg_sp.at[pl.ds(r0 + q * 128, 128)])
    plsc.subcore_barrier()

    base = w * EPW
    if not with_gather:
        ones = jnp.ones((16,), jnp.float32)

        def fill(i, carry):
            for j in range(D // 16):
                rows_w[i, pl.ds(j * 16, 16)] = ones
            return carry

        lax.fori_loop(0, 128, fill, 0)

    if with_gather:
        def batch(i, carry):
            off = w * (EPW // 128) + i * 8
            pltpu.sync_copy(dst_hbm.at[pl.ds(off, 8)], didx_w)
            pltpu.sync_copy(src_hbm.at[pl.ds(off, 8)], sidx_w)
            slot = [(rows_w, sem), (rows_b, sem_b)]
            for q, (rows, sm) in enumerate(slot):
                pltpu.async_copy(x_hbm.at[sidx_w.at[q, :]], rows, sm)
            for j in range(8):
                rows, sm = slot[j % 2]
                pltpu.make_async_copy(
                    x_hbm.at[sidx_w.at[j, :]], rows, sm).wait()
                pltpu.sync_copy(rows, agg_sp.at[didx_w.at[j, :]], add=True)
                if j + 2 < 8:
                    pltpu.async_copy(
                        x_hbm.at[sidx_w.at[j + 2, :]], rows, sm)
            return carry
    else:
        def batch(i, carry):
            off = w * (EPW // 128) + i * 8
            pltpu.sync_copy(dst_hbm.at[pl.ds(off, 8)], didx_w)
            for j in range(8):
                pltpu.sync_copy(rows_w, agg_sp.at[didx_w.at[j, :]], add=True)
            return carry

    lax.fori_loop(0, EPW // 1024, batch, 0)
    plsc.subcore_barrier()

    out_base = c * NR + r0
    for q in range(STRIPE // 128):
        pltpu.sync_copy(agg_sp.at[pl.ds(r0 + q * 128, 128)], rows_w)
        pltpu.sync_copy(rows_w, agg_out.at[pl.ds(out_base + q * 128, 128)])


def _make_seg_sum(with_gather):
    if with_gather:
        scratch = [
            pltpu.VMEM((8, 128), jnp.int32),
            pltpu.VMEM((8, 128), jnp.int32),
            pltpu.VMEM((128, D), jnp.float32),
            pltpu.VMEM((128, D), jnp.float32),
            pltpu.VMEM_SHARED((NR, D), jnp.float32),
            pltpu.SemaphoreType.DMA,
            pltpu.SemaphoreType.DMA,
        ]
    else:
        scratch = [
            pltpu.VMEM((8, 128), jnp.int32),
            pltpu.VMEM((128, D), jnp.float32),
            pltpu.VMEM_SHARED((NR, D), jnp.float32),
            pltpu.SemaphoreType.DMA,
        ]
    return pl.kernel(
        functools.partial(_seg_sum_body, with_gather),
        out_type=jax.ShapeDtypeStruct((NC * NR, D), jnp.float32),
        mesh=plsc.VectorSubcoreMesh(**_MESH),
        scratch_types=scratch,
    )


def _cls_body(h_hbm, e0_hbm, e1_hbm, pred_out,
              i0_w, i1_w, a_v, b_v, out_v, sem):
    c = lax.axis_index("c")
    s = lax.axis_index("s")
    w = s * NC + c
    base = w * ELW
    lanes = lax.iota(jnp.int32, 16)

    def burst(t, carry):
        off = base + t * 128
        pltpu.sync_copy(e0_hbm.at[pl.ds(off, 128)], i0_w)
        pltpu.sync_copy(e1_hbm.at[pl.ds(off, 128)], i1_w)
        d0 = pltpu.async_copy(h_hbm.at[i0_w], a_v, sem)
        d1 = pltpu.async_copy(h_hbm.at[i1_w], b_v, sem)
        d0.wait()
        d1.wait()

        def grp(g, carry2):
            res = jnp.zeros((16,), jnp.float32)
            for r16 in range(16):
                r = g * 16 + r16
                acc = a_v[r, pl.ds(0, 16)] * b_v[r, pl.ds(0, 16)]
                for j in range(1, D // 16):
                    acc = acc + (a_v[r, pl.ds(j * 16, 16)]
                                 * b_v[r, pl.ds(j * 16, 16)])
                for k in (8, 4, 2, 1):
                    acc = acc + jnp.take(acc, lanes ^ k)
                res = jnp.where(lanes == r16, acc, res)
            out_v[pl.ds(g * 16, 16)] = res
            return carry2

        lax.fori_loop(0, 8, grp, 0)
        pltpu.sync_copy(out_v, pred_out.at[pl.ds(off, 128)])
        return carry

    lax.fori_loop(0, ELW // 128, burst, 0)


_cls_kernel = pl.kernel(
    _cls_body,
    out_type=jax.ShapeDtypeStruct((ELPAD,), jnp.float32),
    mesh=plsc.VectorSubcoreMesh(**_MESH),
    scratch_types=[
        pltpu.VMEM((128,), jnp.int32),
        pltpu.VMEM((128,), jnp.int32),
        pltpu.VMEM((128, D), jnp.float32),
        pltpu.VMEM((128, D), jnp.float32),
        pltpu.VMEM((128,), jnp.float32),
        pltpu.SemaphoreType.DMA,
    ],
)


def _tc_body(relu, agg_ref, cnt_ref, x_ref, wl_ref, wr_ref, bl_ref, out_ref):
    aggs = agg_ref[0] + agg_ref[1]
    cnt = cnt_ref[0, :, 0:1] + cnt_ref[1, :, 0:1]
    mean = aggs / jnp.maximum(cnt, 1.0)
    h = lax.dot_general(mean, wl_ref[...], (((1,), (1,)), ((), ())),
                        preferred_element_type=jnp.float32)
    h = h + bl_ref[...]
    h = h + lax.dot_general(x_ref[...], wr_ref[...], (((1,), (1,)), ((), ())),
                            preferred_element_type=jnp.float32)
    if relu:
        h = jnp.maximum(h, 0.0)
    out_ref[...] = h


def _tc_layer(relu, agg, cnt, x, wl, wr, bl):
    R = 1000
    grid = (N // R,)
    return pl.pallas_call(
        functools.partial(_tc_body, relu),
        grid=grid,
        in_specs=[
            pl.BlockSpec((NC, R, D), lambda i: (0, i, 0)),
            pl.BlockSpec((NC, R, D), lambda i: (0, i, 0)),
            pl.BlockSpec((R, D), lambda i: (i, 0)),
            pl.BlockSpec((D, D), lambda i: (0, 0)),
            pl.BlockSpec((D, D), lambda i: (0, 0)),
            pl.BlockSpec((1, D), lambda i: (0, 0)),
        ],
        out_specs=pl.BlockSpec((R, D), lambda i: (i, 0)),
        out_shape=jax.ShapeDtypeStruct((N, D), jnp.float32),
    )(agg, cnt, x, wl, wr, bl)


_seg_sum = _make_seg_sum(True)
_cnt_sum = _make_seg_sum(False)


def kernel(x, edge_index, edge_label_index, Wl1, bl1, Wr1, Wl2, bl2, Wr2):
    ei = edge_index.astype(jnp.int32)
    eli = edge_label_index.astype(jnp.int32)

    pad = EPAD - E
    src = jnp.concatenate([ei[0], jnp.zeros((pad,), jnp.int32)])
    dst = jnp.concatenate([ei[1], jnp.full((pad,), N, jnp.int32)])

    z128 = jnp.zeros((NR, D), jnp.float32)

    src2 = src.reshape(EPAD // 128, 128)
    dst2 = dst.reshape(EPAD // 128, 128)
    agg1 = _seg_sum(src2, dst2, x, z128).reshape(NC, NR, D)
    cnt = _cnt_sum(dst2, z128).reshape(NC, NR, D)
    h1 = _tc_layer(True, agg1, cnt, x, Wl1, Wr1, bl1.reshape(1, D))
    agg2 = _seg_sum(src2, dst2, h1, z128).reshape(NC, NR, D)
    h2 = _tc_layer(False, agg2, cnt, h1, Wl2, Wr2, bl2.reshape(1, D))

    lpad = ELPAD - EL
    e0 = jnp.concatenate([eli[0], jnp.zeros((lpad,), jnp.int32)])
    e1 = jnp.concatenate([eli[1], jnp.zeros((lpad,), jnp.int32)])
    pred = _cls_kernel(h2, e0, e1)
    return pred[:EL]
